# baseline (speedup 1.0000x reference)
import functools

import jax
import jax.numpy as jnp
from jax import lax
from jax.experimental import pallas as pl
from jax.experimental.pallas import tpu as pltpu
from jax.experimental.pallas import tpu_sc as plsc

_NC = 2
_NS = 16
_NW = _NC * _NS
_B = 128
_L = 16
_NBUF = 3


def _sc_aggregate(x, edata, evals, n_pad):
    d = x.shape[1]
    nb = edata.shape[1]
    ngrp = d // _L
    rpt = (-(-n_pad // _NS) + 7) // 8 * 8
    mesh = plsc.VectorSubcoreMesh(core_axis_name="c", subcore_axis_name="s")

    @functools.partial(
        pl.kernel,
        mesh=mesh,
        out_type=jax.ShapeDtypeStruct((_NC, n_pad, d), jnp.float32),
        scratch_types=[pltpu.VMEM((2, _B), jnp.int32) for _ in range(_NBUF)]
          + [pltpu.VMEM((_B,), jnp.float32) for _ in range(_NBUF)]
          + [pltpu.VMEM((_B, d), jnp.float32) for _ in range(_NBUF)]
          + [pltpu.VMEM_SHARED((n_pad, d), jnp.float32)]
          + [pltpu.SemaphoreType.DMA] * (2 * _NBUF),
    )
    def k(x_hbm, edata_hbm, evals_hbm, out_hbm, *rest):
        ebufs = rest[:_NBUF]
        vbufs = rest[_NBUF:2 * _NBUF]
        gbufs = rest[2 * _NBUF:3 * _NBUF]
        acc = rest[3 * _NBUF]
        esem = rest[3 * _NBUF + 1:3 * _NBUF + 1 + _NBUF]
        gsem = rest[3 * _NBUF + 1 + _NBUF:]
        c = lax.axis_index("c")
        s = lax.axis_index("s")
        w = c * _NS + s

        zero_row = jnp.zeros((_L,), jnp.float32)

        def zero_body(i, carry):
            for j in range(ngrp):
                gbufs[0][i, pl.ds(j * _L, _L)] = zero_row
            return carry

        lax.fori_loop(0, _B, zero_body, 0)
        base = jnp.minimum(s * rpt, n_pad - rpt)
        for blk in range(rpt // _B):
            pltpu.sync_copy(gbufs[0], acc.at[pl.ds(base + blk * _B, _B)])
        rem = rpt % _B
        if rem:
            pltpu.sync_copy(gbufs[0].at[pl.ds(0, rem)],
                            acc.at[pl.ds(base + (rpt // _B) * _B, rem)])
        plsc.subcore_barrier()

        def idx_start(b, k_):
            pltpu.async_copy(edata_hbm.at[w, b], ebufs[k_], esem[k_])
            pltpu.async_copy(evals_hbm.at[w, b], vbufs[k_], esem[k_])

        def idx_wait(b, k_):
            pltpu.make_async_copy(
                edata_hbm.at[w, b], ebufs[k_], esem[k_]).wait()
            pltpu.make_async_copy(
                evals_hbm.at[w, b], vbufs[k_], esem[k_]).wait()

        def gather_start(k_):
            pltpu.async_copy(x_hbm.at[ebufs[k_].at[0]], gbufs[k_], gsem[k_])

        def gather_wait(k_):
            pltpu.make_async_copy(
                x_hbm.at[ebufs[k_].at[0]], gbufs[k_], gsem[k_]).wait()

        def scale(k_):
            def scale_body(g, carry2):
                vv = vbufs[k_][pl.ds(g * _L, _L)]
                for l in range(_L):
                    v = vv[l]
                    i = g * _L + l
                    for j in range(ngrp):
                        sl = pl.ds(j * _L, _L)
                        gbufs[k_][i, sl] = gbufs[k_][i, sl] * v
                return carry2

            lax.fori_loop(0, _B // _L, scale_body, 0)

        for k_ in range(_NBUF):
            idx_start(k_, k_)
        for k_ in range(_NBUF - 1):
            idx_wait(k_, k_)
            gather_start(k_)

        def batch_group(p, carry):
            for k_ in range(_NBUF):
                b = p * _NBUF + k_
                k2 = (k_ + _NBUF - 1) % _NBUF
                gather_wait(k_)
                scale(k_)
                pltpu.sync_copy(gbufs[k_], acc.at[ebufs[k_].at[1]], add=True)

                @pl.when(b + _NBUF < nb)
                def _():
                    idx_start(b + _NBUF, k_)

                @pl.when(b + _NBUF - 1 < nb)
                def _():
                    idx_wait(b + _NBUF - 1, k2)
                    gather_start(k2)

            return carry

        lax.fori_loop(0, nb // _NBUF, batch_group, 0)

        plsc.subcore_barrier()
        pltpu.sync_copy(acc.at[pl.ds(base, rpt)], out_hbm.at[c, pl.ds(base, rpt)])

    return k(x, edata, evals)


def _tc_transform(p0, p1, w_mat, bias_row):
    m, d = p0.shape
    bm = 1000

    def body(p0_ref, p1_ref, w_ref, b_ref, o_ref):
        agg = p0_ref[...] + p1_ref[...]
        h = lax.dot_general(agg, w_ref[...], (((1,), (1,)), ((), ())),
                            preferred_element_type=jnp.float32)
        o_ref[...] = jnp.maximum(h + b_ref[...], 0.0)

    return pl.pallas_call(
        body,
        grid=(m // bm,),
        in_specs=[
            pl.BlockSpec((bm, d), lambda i: (i, 0)),
            pl.BlockSpec((bm, d), lambda i: (i, 0)),
            pl.BlockSpec((d, d), lambda i: (0, 0)),
            pl.BlockSpec((1, d), lambda i: (0, 0)),
        ],
        out_specs=pl.BlockSpec((bm, d), lambda i: (i, 0)),
        out_shape=jax.ShapeDtypeStruct((m, d), jnp.float32),
    )(p0, p1, w_mat, bias_row)


def kernel(x, A_indices, A_values, A_shape, W, bias):
    n, d = x.shape
    e = A_values.shape[0]

    chunk = _NW * _B
    nb = (e + chunk - 1) // chunk
    nb = ((nb + _NBUF - 1) // _NBUF) * _NBUF
    e_pad = nb * chunk
    pad = e_pad - e
    rows = jnp.pad(A_indices[0], (0, pad)).reshape(_NW, nb, 1, _B)
    cols = jnp.pad(A_indices[1], (0, pad)).reshape(_NW, nb, 1, _B)
    edata = jnp.concatenate([cols, rows], axis=2)
    evals = jnp.pad(A_values, (0, pad)).reshape(_NW, nb, _B)

    n_pad = ((n + 7) // 8) * 8

    partial = _sc_aggregate(x, edata, evals, n_pad)

    residual = (jnp.asarray(A_shape) - n).astype(jnp.float32)
    bias_row = (bias + residual).reshape(1, d)
    out_full = _tc_transform(partial[0], partial[1], W, bias_row)
    return out_full[:n]


# speedup vs baseline: 2.1326x; 2.1326x over previous
import functools

import jax
import jax.numpy as jnp
from jax import lax
from jax.experimental import pallas as pl
from jax.experimental.pallas import tpu as pltpu
from jax.experimental.pallas import tpu_sc as plsc

_NC = 2
_NS = 16
_NW = _NC * _NS
_B = 128
_SB = 64
_L = 16


def _sc_aggregate(x, packed_cr, vals_r, n_pad):
    d = x.shape[1]
    nb = packed_cr.shape[1]
    nsb = 2 * nb
    ngrp = d // _L
    rpt = n_pad // _NS
    mesh = plsc.VectorSubcoreMesh(core_axis_name="c", subcore_axis_name="s")

    @functools.partial(
        pl.kernel,
        mesh=mesh,
        out_type=jax.ShapeDtypeStruct((_NC, n_pad, d), jnp.float32),
        scratch_types=[
            pltpu.VMEM((nb, _B), jnp.int32),
            pltpu.VMEM((nb, _B), jnp.float32),
            pltpu.VMEM((2 * _SB, d), jnp.float32),
            pltpu.VMEM((_SB,), jnp.int32),
            pltpu.VMEM((_SB,), jnp.int32),
            pltpu.VMEM((_SB,), jnp.int32),
            pltpu.VMEM_SHARED((n_pad, d), jnp.float32),
            pltpu.SemaphoreType.DMA,
            pltpu.SemaphoreType.DMA,
        ],
    )
    def k(x_hbm, packed_hbm, vals_hbm, out_hbm,
          packed_v, vals_v, gbuf, cidx0, cidx1, ridx, acc, sem0, sem1):
        cidx = (cidx0, cidx1)
        gsem = (sem0, sem1)
        c = lax.axis_index("c")
        s = lax.axis_index("s")
        w = c * _NS + s
        mask16 = jnp.full((_L,), 0xFFFF, jnp.int32)
        sh16 = jnp.full((_L,), 16, jnp.int32)

        zero_row = jnp.zeros((_L,), jnp.float32)

        def zero_body(i, carry):
            for j in range(ngrp):
                gbuf[i, pl.ds(j * _L, _L)] = zero_row
            return carry

        lax.fori_loop(0, 2 * _SB, zero_body, 0)
        base = s * rpt
        for blk in range(rpt // (2 * _SB)):
            pltpu.sync_copy(gbuf, acc.at[pl.ds(base + blk * 2 * _SB, 2 * _SB)])

        pltpu.sync_copy(packed_hbm.at[w], packed_v)
        pltpu.sync_copy(vals_hbm.at[w], vals_v)
        plsc.subcore_barrier()

        def gather_issue(b, half):
            for g in range(_SB // _L):
                pv = packed_v[b, pl.ds(half * _SB + g * _L, _L)]
                cidx[half][pl.ds(g * _L, _L)] = lax.bitwise_and(pv, mask16)
            pltpu.async_copy(x_hbm.at[cidx[half]],
                             gbuf.at[pl.ds(half * _SB, _SB)], gsem[half])

        def gather_wait(half):
            pltpu.make_async_copy(x_hbm.at[cidx[half]],
                                  gbuf.at[pl.ds(half * _SB, _SB)],
                                  gsem[half]).wait()

        def scale_unpack(b, half):
            def scale_body(g, carry2):
                off = half * _SB + g * _L
                pv = packed_v[b, pl.ds(off, _L)]
                ridx[pl.ds(g * _L, _L)] = lax.bitwise_and(
                    lax.shift_right_logical(pv, sh16), mask16)
                vv = vals_v[b, pl.ds(off, _L)]
                for l in range(_L):
                    v = vv[l]
                    i = half * _SB + g * _L + l
                    for j in range(ngrp):
                        sl = pl.ds(j * _L, _L)
                        gbuf[i, sl] = gbuf[i, sl] * v
                return carry2

            lax.fori_loop(0, _SB // _L, scale_body, 0)

        gather_issue(0, 0)

        def batch_pair(p, carry):
            for h in range(2):
                nxt_b = p + h
                nxt_half = 1 - h

                @pl.when(nxt_b * 2 + nxt_half < nsb)
                def _():
                    gather_issue(nxt_b, nxt_half)

                gather_wait(h)
                scale_unpack(p, h)
                pltpu.sync_copy(gbuf.at[pl.ds(h * _SB, _SB)],
                                acc.at[ridx], add=True)
            return carry

        lax.fori_loop(0, nb, batch_pair, 0)

        plsc.subcore_barrier()
        pltpu.sync_copy(acc.at[pl.ds(base, rpt)], out_hbm.at[c, pl.ds(base, rpt)])

    return k(x, packed_cr, vals_r)


def _tc_transform(p0, p1, w_mat, bias_row):
    m, d = p0.shape
    bm = 1024

    def body(p0_ref, p1_ref, w_ref, b_ref, o_ref):
        agg = p0_ref[...] + p1_ref[...]
        h = lax.dot_general(agg, w_ref[...], (((1,), (1,)), ((), ())),
                            preferred_element_type=jnp.float32)
        o_ref[...] = jnp.maximum(h + b_ref[...], 0.0)

    return pl.pallas_call(
        body,
        grid=(m // bm,),
        in_specs=[
            pl.BlockSpec((bm, d), lambda i: (i, 0)),
            pl.BlockSpec((bm, d), lambda i: (i, 0)),
            pl.BlockSpec((d, d), lambda i: (0, 0)),
            pl.BlockSpec((1, d), lambda i: (0, 0)),
        ],
        out_specs=pl.BlockSpec((bm, d), lambda i: (i, 0)),
        out_shape=jax.ShapeDtypeStruct((m, d), jnp.float32),
    )(p0, p1, w_mat, bias_row)


def kernel(x, A_indices, A_values, A_shape, W, bias):
    n, d = x.shape
    e = A_values.shape[0]

    chunk = _NW * _B
    nb = (e + chunk - 1) // chunk
    e_pad = nb * chunk
    pad = e_pad - e
    rows = jnp.pad(A_indices[0], (0, pad))
    cols = jnp.pad(A_indices[1], (0, pad))
    packed = (cols | (rows << 16)).reshape(_NW, nb, _B)
    vals = jnp.pad(A_values, (0, pad)).reshape(_NW, nb, _B)

    tile_rows = _NS * _B
    n_pad = ((n + tile_rows - 1) // tile_rows) * tile_rows

    partial = _sc_aggregate(x, packed, vals, n_pad)

    residual = (jnp.asarray(A_shape) - n).astype(jnp.float32)
    bias_row = (bias + residual).reshape(1, d)
    out_full = _tc_transform(partial[0], partial[1], W, bias_row)
    return out_full[:n]


# speedup vs baseline: 4.1617x; 1.9515x over previous
import functools

import jax
import jax.numpy as jnp
from jax import lax
from jax.experimental import pallas as pl
from jax.experimental.pallas import tpu as pltpu
from jax.experimental.pallas import tpu_sc as plsc

_NC = 2
_NS = 16
_NW = _NC * _NS
_SB = 64
_L = 16


def _sc_aggregate(x, a_rows, a_cols, a_val, n_pad):
    d = x.shape[1]
    e_pad = a_val.shape[0]
    ept = e_pad // _NW
    nfull = ept // _SB
    ntail = (ept % _SB) // _L
    ngrp = d // _L
    rpt = (-(-n_pad // _NS) + 7) // 8 * 8
    mesh = plsc.VectorSubcoreMesh(core_axis_name="c", subcore_axis_name="s")

    @functools.partial(
        pl.kernel,
        mesh=mesh,
        out_type=jax.ShapeDtypeStruct((_NC, n_pad, d), jnp.float32),
        scratch_types=[
            pltpu.VMEM((ept,), jnp.int32),
            pltpu.VMEM((ept,), jnp.int32),
            pltpu.VMEM((ept,), jnp.float32),
            pltpu.VMEM((2 * _SB, d), jnp.float32),
            pltpu.VMEM((_SB,), jnp.int32),
            pltpu.VMEM((_SB,), jnp.int32),
            pltpu.VMEM((_SB,), jnp.int32),
            pltpu.VMEM((_L,), jnp.int32),
            pltpu.VMEM((_L,), jnp.int32),
            pltpu.VMEM_SHARED((n_pad, d), jnp.float32),
            pltpu.SemaphoreType.DMA,
            pltpu.SemaphoreType.DMA,
        ],
    )
    def k(x_hbm, arows_hbm, acols_hbm, aval_hbm, out_hbm,
          cols_v, rows_v, vals_v, gbuf, cidx0, cidx1, ridx, ctail, rtail,
          acc, sem0, sem1):
        cidx = (cidx0, cidx1)
        gsem = (sem0, sem1)
        c = lax.axis_index("c")
        s = lax.axis_index("s")
        w = c * _NS + s

        zero_row = jnp.zeros((_L,), jnp.float32)

        def zero_body(i, carry):
            for j in range(ngrp):
                gbuf[i, pl.ds(j * _L, _L)] = zero_row
            return carry

        lax.fori_loop(0, 2 * _SB, zero_body, 0)
        base = jnp.minimum(s * rpt, n_pad - rpt)
        for blk in range(rpt // (2 * _SB)):
            pltpu.sync_copy(gbuf, acc.at[pl.ds(base + blk * 2 * _SB, 2 * _SB)])
        rem = rpt % (2 * _SB)
        if rem:
            pltpu.sync_copy(gbuf.at[pl.ds(0, rem)],
                            acc.at[pl.ds(base + (rpt // (2 * _SB)) * 2 * _SB,
                                         rem)])

        eb = w * ept
        pltpu.sync_copy(acols_hbm.at[pl.ds(eb, ept)], cols_v)
        pltpu.sync_copy(arows_hbm.at[pl.ds(eb, ept)], rows_v)
        pltpu.sync_copy(aval_hbm.at[pl.ds(eb, ept)], vals_v)
        plsc.subcore_barrier()

        def gather_issue(sb, half):
            for g in range(_SB // _L):
                cidx[half][pl.ds(g * _L, _L)] = \
                    cols_v[pl.ds(sb * _SB + g * _L, _L)]
            pltpu.async_copy(x_hbm.at[cidx[half]],
                             gbuf.at[pl.ds(half * _SB, _SB)], gsem[half])

        def gather_wait(half):
            pltpu.make_async_copy(x_hbm.at[cidx[half]],
                                  gbuf.at[pl.ds(half * _SB, _SB)],
                                  gsem[half]).wait()

        def scale_stage(sb, half):
            def scale_body(g, carry2):
                off = sb * _SB + g * _L
                ridx[pl.ds(g * _L, _L)] = rows_v[pl.ds(off, _L)]
                vv = vals_v[pl.ds(off, _L)]
                for l in range(_L):
                    v = vv[l]
                    i = half * _SB + g * _L + l
                    for j in range(ngrp):
                        sl = pl.ds(j * _L, _L)
                        gbuf[i, sl] = gbuf[i, sl] * v
                return carry2

            lax.fori_loop(0, _SB // _L, scale_body, 0)

        def scatter_sync(half):
            pltpu.sync_copy(gbuf.at[pl.ds(half * _SB, _SB)],
                            acc.at[ridx], add=True)

        if nfull:
            gather_issue(0, 0)

            def batch_pair(p, carry):
                for h in range(2):
                    sb = p * 2 + h

                    @pl.when(sb + 1 < nfull)
                    def _():
                        gather_issue(sb + 1, 1 - h)

                    gather_wait(h)
                    scale_stage(sb, h)
                    scatter_sync(h)
                return carry

            lax.fori_loop(0, nfull // 2, batch_pair, 0)
            if nfull % 2:
                sb = nfull - 1
                gather_wait(sb % 2)
                scale_stage(sb, sb % 2)
                scatter_sync(sb % 2)

        for t in range(ntail):
            off = nfull * _SB + t * _L
            ctail[pl.ds(0, _L)] = cols_v[pl.ds(off, _L)]
            pltpu.async_copy(x_hbm.at[ctail], gbuf.at[pl.ds(0, _L)], gsem[0])
            pltpu.make_async_copy(x_hbm.at[ctail], gbuf.at[pl.ds(0, _L)],
                                  gsem[0]).wait()
            rtail[pl.ds(0, _L)] = rows_v[pl.ds(off, _L)]
            vv = vals_v[pl.ds(off, _L)]
            for l in range(_L):
                v = vv[l]
                for j in range(ngrp):
                    sl = pl.ds(j * _L, _L)
                    gbuf[l, sl] = gbuf[l, sl] * v
            pltpu.sync_copy(gbuf.at[pl.ds(0, _L)], acc.at[rtail], add=True)

        plsc.subcore_barrier()
        pltpu.sync_copy(acc.at[pl.ds(base, rpt)], out_hbm.at[c, pl.ds(base, rpt)])

    return k(x, a_rows, a_cols, a_val)


def _tc_transform(partial, w_mat, bias_row):
    _, m, d = partial.shape
    bm = 1000 if m % 1000 == 0 else 8

    def body(p0_ref, p1_ref, w_ref, b_ref, o_ref):
        agg = p0_ref[0] + p1_ref[0]
        h = lax.dot_general(agg, w_ref[...], (((1,), (1,)), ((), ())),
                            preferred_element_type=jnp.float32)
        o_ref[...] = jnp.maximum(h + b_ref[...], 0.0)

    return pl.pallas_call(
        body,
        grid=(m // bm,),
        in_specs=[
            pl.BlockSpec((1, bm, d), lambda i: (0, i, 0)),
            pl.BlockSpec((1, bm, d), lambda i: (1, i, 0)),
            pl.BlockSpec((d, d), lambda i: (0, 0)),
            pl.BlockSpec((1, d), lambda i: (0, 0)),
        ],
        out_specs=pl.BlockSpec((bm, d), lambda i: (i, 0)),
        out_shape=jax.ShapeDtypeStruct((m, d), jnp.float32),
    )(partial, partial, w_mat, bias_row)


def kernel(x, A_indices, A_values, A_shape, W, bias):
    n, d = x.shape
    e = A_values.shape[0]

    group = _L * _NW
    e_pad = ((e + group - 1) // group) * group
    pad = e_pad - e
    a_rows = jnp.pad(A_indices[0], (0, pad))
    a_cols = jnp.pad(A_indices[1], (0, pad))
    a_val = jnp.pad(A_values, (0, pad)) if pad else A_values

    n_pad = ((n + 7) // 8) * 8
    partial = _sc_aggregate(x, a_rows, a_cols, a_val, n_pad)

    residual = (jnp.asarray(A_shape) - n).astype(jnp.float32)
    bias_row = (bias + residual).reshape(1, d)
    out_full = _tc_transform(partial, W, bias_row)
    return out_full[:n]
